# Initial kernel scaffold; baseline (speedup 1.0000x reference)
import jax
import jax.numpy as jnp
from jax.experimental import pallas as pl


def kernel(x, edge_index, Wl0, bl0, Wr0, Wl1, bl1, Wr1):
    raise NotImplementedError("write your pallas kernel here")


# speedup vs baseline: 3.4121x; 3.4121x over previous
import functools

import jax
import jax.numpy as jnp
from jax import lax
from jax.experimental import pallas as pl
from jax.experimental.pallas import tpu as pltpu
from jax.experimental.pallas import tpu_sc as plsc

N = 10000
E = 320000
D = 128

NC = 2
NS = 16
NW = NC * NS
CH = 128
CPW = 80
NCHUNKS = NW * CPW
EP = NCHUNKS * CH
ACC_N = 10240
RPT = ACC_N // NS


def _make_agg(compute_deg):
    out_types = [jax.ShapeDtypeStruct((NC, ACC_N, D), jnp.float32)]
    if compute_deg:
        out_types.append(jax.ShapeDtypeStruct((NC, ACC_N), jnp.float32))
    scratch = [
        pltpu.VMEM((CPW, CH), jnp.int32),
        pltpu.VMEM((CPW, CH), jnp.int32),
        pltpu.VMEM((CH, D), jnp.float32),
        pltpu.VMEM((CH,), jnp.float32),
        pltpu.VMEM_SHARED((ACC_N, D), jnp.float32),
        pltpu.VMEM_SHARED((ACC_N,), jnp.float32),
        pltpu.SemaphoreType.DMA,
    ]
    mesh = plsc.VectorSubcoreMesh(core_axis_name="c", subcore_axis_name="s")

    @functools.partial(
        pl.kernel,
        out_type=out_types if compute_deg else out_types[0],
        mesh=mesh,
        scratch_types=scratch,
    )
    def agg(x_hbm, src_hbm, dst_hbm, z2_hbm, z1_hbm, one_hbm, *rest):
        if compute_deg:
            (acc_out, deg_out, src_v, dst_v, rows_v, ones_v,
             acc_sh, deg_sh, sem) = rest
        else:
            deg_out = None
            (acc_out, src_v, dst_v, rows_v, ones_v,
             acc_sh, deg_sh, sem) = rest
        c = lax.axis_index("c")
        s = lax.axis_index("s")
        wid = s * NC + c

        @pl.when(s == 0)
        def _():
            pltpu.sync_copy(z2_hbm, acc_sh)
            if compute_deg:
                pltpu.sync_copy(z1_hbm, deg_sh)

        base = wid * CPW
        pltpu.sync_copy(src_hbm.at[pl.ds(base, CPW)], src_v)
        pltpu.sync_copy(dst_hbm.at[pl.ds(base, CPW)], dst_v)
        if compute_deg:
            pltpu.sync_copy(one_hbm, ones_v)
        plsc.subcore_barrier()

        def body(j, carry):
            pltpu.async_copy(x_hbm.at[src_v.at[j]], rows_v, sem).wait()
            pltpu.sync_copy(rows_v, acc_sh.at[dst_v.at[j]], add=True)
            if compute_deg:
                pltpu.sync_copy(ones_v, deg_sh.at[dst_v.at[j]], add=True)
            return carry

        lax.fori_loop(0, CPW, body, 0)
        plsc.subcore_barrier()

        r0 = s * RPT
        pltpu.sync_copy(acc_sh.at[pl.ds(r0, RPT)], acc_out.at[c, pl.ds(r0, RPT)])
        if compute_deg:
            pltpu.sync_copy(deg_sh.at[pl.ds(r0, RPT)],
                            deg_out.at[c, pl.ds(r0, RPT)])

    return agg


_agg_deg = _make_agg(True)
_agg_nodeg = _make_agg(False)


def _dense_body(p_ref, deg_ref, x_ref, wl_ref, bl_ref, wr_ref, o_ref):
    agg = p_ref[0, :N, :] + p_ref[1, :N, :]
    deg = deg_ref[0, :N] + deg_ref[1, :N]
    inv = 1.0 / jnp.maximum(deg, 1.0)
    agg = agg * inv[:, None]
    y = jnp.dot(agg, wl_ref[...], preferred_element_type=jnp.float32)
    y = y + bl_ref[...]
    y = y + jnp.dot(x_ref[...], wr_ref[...], preferred_element_type=jnp.float32)
    o_ref[...] = jnp.maximum(y, 0.0)


def _dense(p, degp, xin, wlT, bl2, wrT):
    return pl.pallas_call(
        _dense_body,
        out_shape=jax.ShapeDtypeStruct((N, D), jnp.float32),
    )(p, degp, xin, wlT, bl2, wrT)


def kernel(x, edge_index, Wl0, bl0, Wr0, Wl1, bl1, Wr1):
    src = edge_index[0]
    dst = edge_index[1]
    pad = EP - E
    src_p = jnp.concatenate(
        [src, jnp.zeros((pad,), jnp.int32)]).reshape(NCHUNKS, CH)
    dst_p = jnp.concatenate(
        [dst, jnp.full((pad,), N, jnp.int32)]).reshape(NCHUNKS, CH)
    z2 = jnp.zeros((ACC_N, D), jnp.float32)
    z1 = jnp.zeros((ACC_N,), jnp.float32)
    ones = jnp.ones((CH,), jnp.float32)

    p0, degp = _agg_deg(x, src_p, dst_p, z2, z1, ones)
    h = _dense(p0, degp, x, Wl0.T, bl0.reshape(1, D), Wr0.T)
    p1 = _agg_nodeg(h, src_p, dst_p, z2, z1, ones)
    out = _dense(p1, degp, h, Wl1.T, bl1.reshape(1, D), Wr1.T)
    return out


# speedup vs baseline: 3.6981x; 1.0838x over previous
import functools

import jax
import jax.numpy as jnp
from jax import lax
from jax.experimental import pallas as pl
from jax.experimental.pallas import tpu as pltpu
from jax.experimental.pallas import tpu_sc as plsc

N = 10000
E = 320000
D = 128

NC = 2
NS = 16
NW = NC * NS
CH = 128
CPW = 80
NCHUNKS = NW * CPW
EP = NCHUNKS * CH
ACC_N = 10240
RPT = ACC_N // NS


def _make_agg(compute_deg):
    out_types = [jax.ShapeDtypeStruct((NC, ACC_N, D), jnp.float32)]
    if compute_deg:
        out_types.append(jax.ShapeDtypeStruct((NC, ACC_N), jnp.float32))
    scratch = [
        pltpu.VMEM((CPW, CH), jnp.int32),
        pltpu.VMEM((1, CH), jnp.int32),
        pltpu.VMEM((1, CH), jnp.int32),
        pltpu.VMEM((CH, D), jnp.float32),
        pltpu.VMEM((CH, D), jnp.float32),
        pltpu.VMEM((CH,), jnp.float32),
        pltpu.VMEM_SHARED((ACC_N, D), jnp.float32),
        pltpu.VMEM_SHARED((ACC_N,), jnp.float32),
        pltpu.SemaphoreType.DMA,
        pltpu.SemaphoreType.DMA,
        pltpu.SemaphoreType.DMA,
        pltpu.SemaphoreType.DMA,
    ]
    mesh = plsc.VectorSubcoreMesh(core_axis_name="c", subcore_axis_name="s")

    @functools.partial(
        pl.kernel,
        out_type=out_types if compute_deg else out_types[0],
        mesh=mesh,
        scratch_types=scratch,
    )
    def agg(x_hbm, src_hbm, dst_hbm, z2_hbm, z1_hbm, one_hbm, *rest):
        if compute_deg:
            (acc_out, deg_out, src_v, dst_a, dst_b, rows_a, rows_b, ones_v,
             acc_sh, deg_sh, sem_a, sem_b, sem_da, sem_db) = rest
        else:
            deg_out = None
            (acc_out, src_v, dst_a, dst_b, rows_a, rows_b, ones_v,
             acc_sh, deg_sh, sem_a, sem_b, sem_da, sem_db) = rest
        c = lax.axis_index("c")
        s = lax.axis_index("s")
        wid = s * NC + c

        @pl.when(s == 0)
        def _():
            pltpu.sync_copy(z2_hbm, acc_sh)
            if compute_deg:
                pltpu.sync_copy(z1_hbm, deg_sh)

        base = wid * CPW
        pltpu.sync_copy(src_hbm.at[pl.ds(base, CPW)], src_v)
        if compute_deg:
            pltpu.sync_copy(one_hbm, ones_v)
        plsc.subcore_barrier()

        pltpu.async_copy(dst_hbm.at[base], dst_a, sem_da)
        pltpu.async_copy(dst_hbm.at[base + 1], dst_b, sem_db)
        pltpu.async_copy(x_hbm.at[src_v.at[0]], rows_a, sem_a)
        last = CPW // 2 - 1

        def body(i, carry):
            j0 = 2 * i
            j1 = j0 + 1
            pltpu.make_async_copy(x_hbm.at[src_v.at[j0]], rows_a, sem_a).wait()
            pltpu.async_copy(x_hbm.at[src_v.at[j1]], rows_b, sem_b)
            pltpu.make_async_copy(dst_hbm.at[base], dst_a, sem_da).wait()
            pltpu.sync_copy(rows_a, acc_sh.at[dst_a.at[0]], add=True)
            if compute_deg:
                pltpu.sync_copy(ones_v, deg_sh.at[dst_a.at[0]], add=True)

            @pl.when(i < last)
            def _():
                pltpu.async_copy(dst_hbm.at[base + j0 + 2], dst_a, sem_da)

            pltpu.make_async_copy(x_hbm.at[src_v.at[j1]], rows_b, sem_b).wait()

            @pl.when(i < last)
            def _():
                pltpu.async_copy(x_hbm.at[src_v.at[j0 + 2]], rows_a, sem_a)

            pltpu.make_async_copy(dst_hbm.at[base + 1], dst_b, sem_db).wait()
            pltpu.sync_copy(rows_b, acc_sh.at[dst_b.at[0]], add=True)
            if compute_deg:
                pltpu.sync_copy(ones_v, deg_sh.at[dst_b.at[0]], add=True)

            @pl.when(i < last)
            def _():
                pltpu.async_copy(dst_hbm.at[base + j1 + 2], dst_b, sem_db)

            return carry

        lax.fori_loop(0, CPW // 2, body, 0)
        plsc.subcore_barrier()

        r0 = s * RPT
        pltpu.sync_copy(acc_sh.at[pl.ds(r0, RPT)], acc_out.at[c, pl.ds(r0, RPT)])
        if compute_deg:
            pltpu.sync_copy(deg_sh.at[pl.ds(r0, RPT)],
                            deg_out.at[c, pl.ds(r0, RPT)])

    return agg


_agg_deg = _make_agg(True)
_agg_nodeg = _make_agg(False)


def _dense_body(p_ref, deg_ref, x_ref, wl_ref, bl_ref, wr_ref, o_ref):
    agg = p_ref[0, :N, :] + p_ref[1, :N, :]
    deg = deg_ref[0, :N] + deg_ref[1, :N]
    inv = 1.0 / jnp.maximum(deg, 1.0)
    agg = agg * inv[:, None]
    y = jnp.dot(agg, wl_ref[...], preferred_element_type=jnp.float32)
    y = y + bl_ref[...]
    y = y + jnp.dot(x_ref[...], wr_ref[...], preferred_element_type=jnp.float32)
    o_ref[...] = jnp.maximum(y, 0.0)


def _dense(p, degp, xin, wlT, bl2, wrT):
    return pl.pallas_call(
        _dense_body,
        out_shape=jax.ShapeDtypeStruct((N, D), jnp.float32),
    )(p, degp, xin, wlT, bl2, wrT)


def kernel(x, edge_index, Wl0, bl0, Wr0, Wl1, bl1, Wr1):
    src = edge_index[0]
    dst = edge_index[1]
    pad = EP - E
    src_p = jnp.concatenate(
        [src, jnp.zeros((pad,), jnp.int32)]).reshape(NCHUNKS, CH)
    dst_p = jnp.concatenate(
        [dst, jnp.full((pad,), N, jnp.int32)]).reshape(NCHUNKS, 1, CH)
    z2 = jnp.zeros((ACC_N, D), jnp.float32)
    z1 = jnp.zeros((ACC_N,), jnp.float32)
    ones = jnp.ones((CH,), jnp.float32)

    p0, degp = _agg_deg(x, src_p, dst_p, z2, z1, ones)
    h = _dense(p0, degp, x, Wl0.T, bl0.reshape(1, D), Wr0.T)
    p1 = _agg_nodeg(h, src_p, dst_p, z2, z1, ones)
    out = _dense(p1, degp, h, Wl1.T, bl1.reshape(1, D), Wr1.T)
    return out
